# baseline (speedup 1.0000x reference)
import functools

import jax
import jax.numpy as jnp
from jax import lax
from jax.experimental import pallas as pl
from jax.experimental.pallas import tpu as pltpu
from jax.experimental.pallas import tpu_sc as plsc

_OUT_DIM = 32
_BATCH = 4096
_SEQ = 200
_VQ = 250000
_NC = 2
_NS = 16
_NW = _NC * _NS
_BW = _BATCH // _NW
_NBUF = 4
_NT = 2
_NG = _SEQ // _NBUF


def _make_fused():
    mesh = plsc.VectorSubcoreMesh(core_axis_name="c", subcore_axis_name="s")

    @functools.partial(
        pl.kernel,
        out_type=jax.ShapeDtypeStruct((_SEQ, _OUT_DIM, _BATCH), jnp.float32),
        mesh=mesh,
        scratch_types=[
            pltpu.VMEM((_SEQ, _BW), jnp.int32),
            pltpu.VMEM((_SEQ, _BW), jnp.int32),
            pltpu.VMEM((_NBUF, _BW, 128), jnp.float32),
            pltpu.VMEM((_NT, _OUT_DIM, _BW), jnp.float32),
            [pltpu.SemaphoreType.DMA] * _NBUF,
            [pltpu.SemaphoreType.DMA] * _NT,
        ],
        compiler_params=pltpu.CompilerParams(
            use_tc_tiling_on_sc=True, needs_layout_passes=False),
    )
    def fused_kernel(idxT_hbm, tableQ_hbm, outT_hbm, qo_v, g_v, R, T, sg, st):
        wid = lax.axis_index("s") * _NC + lax.axis_index("c")
        base = wid * _BW

        pltpu.sync_copy(idxT_hbm.at[:, pl.ds(base, _BW)], qo_v)

        def prep(s, c):
            for k8 in range(_BW // 16):
                sl = pl.ds(k8 * 16, 16)
                v = qo_v[s, sl]
                g_v[s, sl] = v >> 2
                qo_v[s, sl] = (v & 3) << 5
            return c

        lax.fori_loop(0, _SEQ, prep, 0)

        def gather_copy(s, b):
            return pltpu.make_async_copy(
                tableQ_hbm.at[g_v.at[s]], R.at[b], sg[b])

        def store_copy(s, tb):
            return pltpu.make_async_copy(
                T.at[tb],
                outT_hbm.at[s, pl.ds(0, _OUT_DIM), pl.ds(base, _BW)],
                st[tb])

        iot = lax.iota(jnp.int32, 16)

        zero16 = iot * 0

        def transpose_s(s, b, tb):
            @plsc.parallel_loop(0, _BW // 16, 1, unroll=8)
            def _(grp):
                qo = qo_v[s, pl.ds(grp * 16, 16)]
                flat = (iot + grp * 16) * 128 + qo
                for d0 in range(0, _OUT_DIM, 8):
                    vals = [
                        plsc.load_gather(R.at[b], [zero16, flat + (d0 + i)])
                        for i in range(8)
                    ]
                    for i in range(8):
                        T[tb, d0 + i, pl.ds(grp * 16, 16)] = vals[i]

        for b in range(_NBUF):
            gather_copy(b, b).start()
        for b in range(_NBUF):
            s = b
            gather_copy(s, b).wait()
            if s >= _NT:
                store_copy(s - _NT, b % _NT).wait()
            transpose_s(s, b, b % _NT)
            store_copy(s, b % _NT).start()
            gather_copy(s + _NBUF, b).start()

        def turn(g, c):
            s0 = g * _NBUF
            for b in range(_NBUF):
                s = s0 + b
                gather_copy(s, b).wait()
                store_copy(s, b % _NT).wait()
                transpose_s(s, b, b % _NT)
                store_copy(s, b % _NT).start()

                @pl.when(s + _NBUF < _SEQ)
                def _():
                    gather_copy(s + _NBUF, b).start()

            return c

        lax.fori_loop(1, _NG, turn, 0)

        store_copy(_SEQ - 2, 0).wait()
        store_copy(_SEQ - 1, 1).wait()

    return fused_kernel


_fused = _make_fused()


@jax.jit
def kernel(inputs, embeddings):
    idxT = jnp.swapaxes(inputs.astype(jnp.int32), 0, 1)
    tableQ = jnp.reshape(embeddings, (_VQ, 128))
    outT = _fused(idxT, tableQ)
    return jnp.transpose(outT, (2, 0, 1))


# speedup vs baseline: 1.0314x; 1.0314x over previous
import functools

import jax
import jax.numpy as jnp
from jax import lax
from jax.experimental import pallas as pl
from jax.experimental.pallas import tpu as pltpu
from jax.experimental.pallas import tpu_sc as plsc

_OUT_DIM = 32
_BATCH = 4096
_SEQ = 200
_NC = 2
_NS = 16
_NW = _NC * _NS
_BW = _BATCH // _NW
_NBUF = 4
_NT = 2
_NG = _SEQ // _NBUF


def _make_fused():
    mesh = plsc.VectorSubcoreMesh(core_axis_name="c", subcore_axis_name="s")

    @functools.partial(
        pl.kernel,
        out_type=jax.ShapeDtypeStruct(
            (_SEQ, _OUT_DIM // 8, _BATCH // 128, 8, 128), jnp.float32),
        mesh=mesh,
        scratch_types=[
            pltpu.VMEM((_SEQ, _BW), jnp.int32),
            pltpu.VMEM((_NBUF, _BW, _OUT_DIM), jnp.float32),
            pltpu.VMEM((_NT, _OUT_DIM, _BW), jnp.float32),
            [pltpu.SemaphoreType.DMA] * _NBUF,
            [pltpu.SemaphoreType.DMA] * _NT,
        ],
        compiler_params=pltpu.CompilerParams(
            use_tc_tiling_on_sc=False, needs_layout_passes=False),
    )
    def fused_kernel(idxT_hbm, table_hbm, out_hbm, idx_v, R, T, sg, st):
        wid = lax.axis_index("s") * _NC + lax.axis_index("c")
        base = wid * _BW
        tj = wid

        pltpu.sync_copy(idxT_hbm.at[:, pl.ds(base, _BW)], idx_v)

        def gather_copy(s, b):
            return pltpu.make_async_copy(
                table_hbm.at[idx_v.at[s]], R.at[b], sg[b])

        def store_copy(s, tb):
            return [
                pltpu.make_async_copy(
                    T.at[tb, pl.ds(ti * 8, 8)],
                    out_hbm.at[s, ti, tj],
                    st[tb])
                for ti in range(_OUT_DIM // 8)
            ]

        iot = lax.iota(jnp.int32, 16)
        zero16 = iot * 0

        def transpose_s(s, b, tb):
            @plsc.parallel_loop(0, _BW // 16, 1, unroll=4)
            def _(grp):
                flat = (iot + grp * 16) * _OUT_DIM
                for d0 in range(0, _OUT_DIM, 8):
                    vals = [
                        plsc.load_gather(R.at[b], [zero16, flat + (d0 + i)])
                        for i in range(8)
                    ]
                    for i in range(8):
                        T[tb, d0 + i, pl.ds(grp * 16, 16)] = vals[i]

        for b in range(_NBUF):
            gather_copy(b, b).start()
        for b in range(_NBUF):
            s = b
            gather_copy(s, b).wait()
            if s >= _NT:
                for c in store_copy(s - _NT, b % _NT):
                    c.wait()
            transpose_s(s, b, b % _NT)
            for c in store_copy(s, b % _NT):
                c.start()
            gather_copy(s + _NBUF, b).start()

        def turn(g, c0):
            s0 = g * _NBUF
            for b in range(_NBUF):
                s = s0 + b
                gather_copy(s, b).wait()
                for c in store_copy(s, b % _NT):
                    c.wait()
                transpose_s(s, b, b % _NT)
                for c in store_copy(s, b % _NT):
                    c.start()

                @pl.when(s + _NBUF < _SEQ)
                def _():
                    gather_copy(s + _NBUF, b).start()

            return c0

        lax.fori_loop(1, _NG, turn, 0)

        for c in store_copy(_SEQ - 2, 0):
            c.wait()
        for c in store_copy(_SEQ - 1, 1):
            c.wait()

    return fused_kernel


_fused = _make_fused()


@jax.jit
def kernel(inputs, embeddings):
    idxT = jnp.swapaxes(inputs.astype(jnp.int32), 0, 1)
    out5 = _fused(idxT, embeddings)
    r = jnp.transpose(out5, (2, 4, 0, 1, 3))
    return jnp.reshape(r, (_BATCH, _SEQ, _OUT_DIM))


# speedup vs baseline: 1.0337x; 1.0022x over previous
import functools

import jax
import jax.numpy as jnp
from jax import lax
from jax.experimental import pallas as pl
from jax.experimental.pallas import tpu as pltpu
from jax.experimental.pallas import tpu_sc as plsc

_OUT_DIM = 32
_BATCH = 4096
_SEQ = 200
_NC = 2
_NS = 16
_NW = _NC * _NS
_BW = _BATCH // _NW
_NBUF = 4
_NT = 2
_NG = _SEQ // _NBUF


def _make_fused():
    mesh = plsc.VectorSubcoreMesh(core_axis_name="c", subcore_axis_name="s")

    @functools.partial(
        pl.kernel,
        out_type=jax.ShapeDtypeStruct(
            (_SEQ, _OUT_DIM // 8, _BATCH // 128, 8, 128), jnp.float32),
        mesh=mesh,
        scratch_types=[
            pltpu.VMEM((_SEQ, _BW), jnp.int32),
            pltpu.VMEM((_NBUF, _BW, _OUT_DIM), jnp.float32),
            pltpu.VMEM((_NT, _OUT_DIM // 8, 1, 8, _BW), jnp.float32),
            [pltpu.SemaphoreType.DMA] * _NBUF,
            [pltpu.SemaphoreType.DMA] * _NT,
        ],
        compiler_params=pltpu.CompilerParams(
            use_tc_tiling_on_sc=False, needs_layout_passes=False),
    )
    def fused_kernel(idxT_hbm, table_hbm, out_hbm, idx_v, R, T, sg, st):
        wid = lax.axis_index("s") * _NC + lax.axis_index("c")
        base = wid * _BW
        tj = wid

        pltpu.sync_copy(idxT_hbm.at[:, pl.ds(base, _BW)], idx_v)

        def gather_copy(s, b):
            return pltpu.make_async_copy(
                table_hbm.at[idx_v.at[s]], R.at[b], sg[b])

        def store_copy(s, tb):
            return pltpu.make_async_copy(
                T.at[tb],
                out_hbm.at[s, pl.ds(0, _OUT_DIM // 8), pl.ds(tj, 1)],
                st[tb])

        iot = lax.iota(jnp.int32, 16)
        zero16 = iot * 0

        def transpose_s(s, b, tb):
            @plsc.parallel_loop(0, _BW // 16, 1, unroll=4)
            def _(grp):
                flat = (iot + grp * 16) * _OUT_DIM
                for d0 in range(0, _OUT_DIM, 8):
                    vals = [
                        plsc.load_gather(R.at[b], [zero16, flat + (d0 + i)])
                        for i in range(8)
                    ]
                    for i in range(8):
                        d = d0 + i
                        T[tb, d // 8, 0, d % 8, pl.ds(grp * 16, 16)] = vals[i]

        for b in range(_NBUF):
            gather_copy(b, b).start()
        for b in range(_NBUF):
            s = b
            gather_copy(s, b).wait()
            if s >= _NT:
                store_copy(s - _NT, b % _NT).wait()
            transpose_s(s, b, b % _NT)
            store_copy(s, b % _NT).start()
            gather_copy(s + _NBUF, b).start()

        def turn(g, c0):
            s0 = g * _NBUF
            for b in range(_NBUF):
                s = s0 + b
                gather_copy(s, b).wait()
                store_copy(s, b % _NT).wait()
                transpose_s(s, b, b % _NT)
                store_copy(s, b % _NT).start()

                @pl.when(s + _NBUF < _SEQ)
                def _():
                    gather_copy(s + _NBUF, b).start()

            return c0

        lax.fori_loop(1, _NG, turn, 0)

        store_copy(_SEQ - 2, 0).wait()
        store_copy(_SEQ - 1, 1).wait()

    return fused_kernel


_fused = _make_fused()


@jax.jit
def kernel(inputs, embeddings):
    idxT = jnp.swapaxes(inputs.astype(jnp.int32), 0, 1)
    out5 = _fused(idxT, embeddings)
    r = jnp.transpose(out5, (2, 4, 0, 1, 3))
    return jnp.reshape(r, (_BATCH, _SEQ, _OUT_DIM))
